# baseline (speedup 1.0000x reference)
import jax
import jax.numpy as jnp
from jax import lax
from jax.experimental import pallas as pl
from jax.experimental.pallas import tpu as pltpu
from jax.experimental.pallas import tpu_sc as plsc

N_EDGES = 800000
EMBED_DIM = 64
NPAIR = N_EDGES // 2
NUM_CORES = 2
NUM_SUBCORES = 16
NW = NUM_CORES * NUM_SUBCORES
PER_W = NPAIR // NW
STAGE = 12504
CHUNK = 200
NCHUNK = 64
NBUF = 4

TR_ROWS = 3200
TR_NB = NPAIR // TR_ROWS


def _sc_body(w2_hbm, idx_hbm, out_hbm, idx_v, table_s,
             b0, b1, b2, b3, g0, g1, g2, g3, w0, w1, w2, w3):
    cid = lax.axis_index("c")
    sid = lax.axis_index("s")
    wid = sid * NUM_CORES + cid

    raw = wid * PER_W
    base = raw + (raw % 8)
    nraw = (wid + 1) * PER_W
    nbase = nraw + (nraw % 8)
    size = nbase - base
    base = pl.multiple_of(base, 8)

    bufs = (b0, b1, b2, b3)
    gsems = (g0, g1, g2, g3)
    wsems = (w0, w1, w2, w3)

    @pl.when(sid == 0)
    def _stage_table():
        pltpu.sync_copy(w2_hbm, table_s)

    pltpu.sync_copy(idx_hbm.at[pl.ds(base, STAGE)], idx_v)
    plsc.subcore_barrier()

    def chunk_off(j):
        return pl.multiple_of(lax.min(j * CHUNK, size - CHUNK), 8)

    def start_gather(j, b):
        pltpu.async_copy(
            table_s.at[idx_v.at[pl.ds(chunk_off(j), CHUNK)]], bufs[b], gsems[b]
        )

    def wait_gather(b):
        pltpu.make_async_copy(
            table_s.at[idx_v.at[pl.ds(0, CHUNK)]], bufs[b], gsems[b]
        ).wait()

    def start_write(j, b):
        pltpu.async_copy(
            bufs[b], out_hbm.at[pl.ds(base + chunk_off(j), CHUNK)], wsems[b]
        )

    def wait_write(b):
        pltpu.make_async_copy(
            bufs[b], out_hbm.at[pl.ds(base, CHUNK)], wsems[b]
        ).wait()

    for b in range(NBUF):
        start_gather(b, b)

    def step(t, carry):
        for b in range(NBUF):
            wait_gather(b)
            start_write(NBUF * t + b, b)
        for b in range(NBUF):
            wait_write(b)
            start_gather(NBUF * t + NBUF + b, b)
        return carry

    lax.fori_loop(0, (NCHUNK - NBUF) // NBUF, step, 0)

    for b in range(NBUF):
        wait_gather(b)
        start_write(NCHUNK - NBUF + b, b)
    for b in range(NBUF):
        wait_write(b)


def _sc_gather(pidx_padded, w2):
    mesh = plsc.VectorSubcoreMesh(core_axis_name="c", subcore_axis_name="s")
    k = pl.kernel(
        _sc_body,
        out_type=jax.ShapeDtypeStruct((NPAIR, 2 * EMBED_DIM), jnp.float32),
        mesh=mesh,
        scratch_types=[
            pltpu.VMEM((STAGE,), jnp.int32),
            pltpu.VMEM_SHARED((16, 2 * EMBED_DIM), jnp.float32),
            pltpu.VMEM((CHUNK, 2 * EMBED_DIM), jnp.float32),
            pltpu.VMEM((CHUNK, 2 * EMBED_DIM), jnp.float32),
            pltpu.VMEM((CHUNK, 2 * EMBED_DIM), jnp.float32),
            pltpu.VMEM((CHUNK, 2 * EMBED_DIM), jnp.float32),
            pltpu.SemaphoreType.DMA,
            pltpu.SemaphoreType.DMA,
            pltpu.SemaphoreType.DMA,
            pltpu.SemaphoreType.DMA,
            pltpu.SemaphoreType.DMA,
            pltpu.SemaphoreType.DMA,
            pltpu.SemaphoreType.DMA,
            pltpu.SemaphoreType.DMA,
        ],
        compiler_params=pltpu.CompilerParams(
            use_tc_tiling_on_sc=False, needs_layout_passes=False
        ),
    )
    return k(w2, pidx_padded)


def _tr_body(x_ref, y_ref):
    h = pl.program_id(1)
    eye = jax.lax.broadcasted_iota(jnp.int32, (EMBED_DIM, EMBED_DIM), 0)
    eyef = jnp.where(
        eye == jax.lax.broadcasted_iota(jnp.int32, (EMBED_DIM, EMBED_DIM), 1),
        1.0,
        0.0,
    ).astype(jnp.float32)

    def tr_mxu(xh):
        return jax.lax.dot_general(
            eyef, xh, (((1,), (1,)), ((), ())),
            precision=jax.lax.Precision.HIGHEST,
            preferred_element_type=jnp.float32,
        )

    def tr_xlu(xh):
        return jnp.transpose(xh, (1, 0))

    y_ref[...] = jnp.where(
        h == 0, tr_mxu(x_ref[:, :EMBED_DIM]), tr_xlu(x_ref[:, EMBED_DIM:])
    )


def _tc_transpose(x):
    return pl.pallas_call(
        _tr_body,
        grid=(TR_NB, 2),
        in_specs=[pl.BlockSpec((TR_ROWS, 2 * EMBED_DIM), lambda r, h: (r, 0))],
        out_specs=pl.BlockSpec((EMBED_DIM, TR_ROWS), lambda r, h: (0, h * TR_NB + r)),
        out_shape=jax.ShapeDtypeStruct((EMBED_DIM, N_EDGES), jnp.float32),
    )(x)


def kernel(edge_attr, weight):
    idx = edge_attr.astype(jnp.int32)
    pidx = idx[:NPAIR] * 4 + idx[NPAIR:]
    pidx_padded = jnp.zeros((NPAIR + 128,), jnp.int32).at[:NPAIR].set(pidx)
    w2 = jnp.concatenate(
        [jnp.repeat(weight, 4, axis=0), jnp.tile(weight, (4, 1))], axis=1
    )
    x = _sc_gather(pidx_padded, w2)
    y = _tc_transpose(x)
    return y.T


# speedup vs baseline: 1.2878x; 1.2878x over previous
import jax
import jax.numpy as jnp
from jax import lax
from jax.experimental import pallas as pl
from jax.experimental.pallas import tpu as pltpu
from jax.experimental.pallas import tpu_sc as plsc

N_EDGES = 800000
EMBED_DIM = 64
NPAIR = N_EDGES // 2
NUM_CORES = 2
NUM_SUBCORES = 16
NW = NUM_CORES * NUM_SUBCORES
PER_W = NPAIR // NW
STAGE = 12504
CHUNK = 200
NCHUNK = 64
NBUF = 4

TR_ROWS = 3200
TR_NB = NPAIR // TR_ROWS


def _sc_body(w2_hbm, idx_hbm, out_hbm, idx_v, table_s,
             b0, b1, b2, b3, g0, g1, g2, g3, w0, w1, w2, w3):
    cid = lax.axis_index("c")
    sid = lax.axis_index("s")
    wid = sid * NUM_CORES + cid

    raw = wid * PER_W
    base = raw + (raw % 8)
    nraw = (wid + 1) * PER_W
    nbase = nraw + (nraw % 8)
    size = nbase - base
    base = pl.multiple_of(base, 8)

    bufs = (b0, b1, b2, b3)
    gsems = (g0, g1, g2, g3)
    wsems = (w0, w1, w2, w3)

    @pl.when(sid == 0)
    def _stage_table():
        pltpu.sync_copy(w2_hbm, table_s)

    pltpu.sync_copy(idx_hbm.at[pl.ds(base, STAGE)], idx_v)
    plsc.subcore_barrier()

    def chunk_off(j):
        return pl.multiple_of(lax.min(j * CHUNK, size - CHUNK), 8)

    def start_gather(j, b):
        pltpu.async_copy(
            table_s.at[idx_v.at[pl.ds(chunk_off(j), CHUNK)]], bufs[b], gsems[b]
        )

    def wait_gather(b):
        pltpu.make_async_copy(
            table_s.at[idx_v.at[pl.ds(0, CHUNK)]], bufs[b], gsems[b]
        ).wait()

    def start_write(j, b):
        pltpu.async_copy(
            bufs[b], out_hbm.at[pl.ds(base + chunk_off(j), CHUNK)], wsems[b]
        )

    def wait_write(b):
        pltpu.make_async_copy(
            bufs[b], out_hbm.at[pl.ds(base, CHUNK)], wsems[b]
        ).wait()

    for b in range(NBUF):
        start_gather(b, b)

    def step(t, carry):
        for b in range(NBUF):
            wait_gather(b)
            start_write(NBUF * t + b, b)
        for b in range(NBUF):
            wait_write(b)
            start_gather(NBUF * t + NBUF + b, b)
        return carry

    lax.fori_loop(0, (NCHUNK - NBUF) // NBUF, step, 0)

    for b in range(NBUF):
        wait_gather(b)
        start_write(NCHUNK - NBUF + b, b)
    for b in range(NBUF):
        wait_write(b)


def _sc_gather(pidx_padded, w2):
    mesh = plsc.VectorSubcoreMesh(core_axis_name="c", subcore_axis_name="s")
    k = pl.kernel(
        _sc_body,
        out_type=jax.ShapeDtypeStruct((NPAIR, 2 * EMBED_DIM), jnp.float32),
        mesh=mesh,
        scratch_types=[
            pltpu.VMEM((STAGE,), jnp.int32),
            pltpu.VMEM_SHARED((16, 2 * EMBED_DIM), jnp.float32),
            pltpu.VMEM((CHUNK, 2 * EMBED_DIM), jnp.float32),
            pltpu.VMEM((CHUNK, 2 * EMBED_DIM), jnp.float32),
            pltpu.VMEM((CHUNK, 2 * EMBED_DIM), jnp.float32),
            pltpu.VMEM((CHUNK, 2 * EMBED_DIM), jnp.float32),
            pltpu.SemaphoreType.DMA,
            pltpu.SemaphoreType.DMA,
            pltpu.SemaphoreType.DMA,
            pltpu.SemaphoreType.DMA,
            pltpu.SemaphoreType.DMA,
            pltpu.SemaphoreType.DMA,
            pltpu.SemaphoreType.DMA,
            pltpu.SemaphoreType.DMA,
        ],
        compiler_params=pltpu.CompilerParams(
            use_tc_tiling_on_sc=False, needs_layout_passes=False
        ),
    )
    return k(w2, pidx_padded)


def _tr_body(x_ref, y_ref, t_ref):
    h = pl.program_id(1)

    @pl.when(h == 0)
    def _full_transpose():
        t_ref[...] = jnp.transpose(x_ref[...], (1, 0))

    y_ref[...] = t_ref[pl.ds(h * EMBED_DIM, EMBED_DIM), :]


def _tc_transpose(x):
    return pl.pallas_call(
        _tr_body,
        grid=(TR_NB, 2),
        in_specs=[pl.BlockSpec((TR_ROWS, 2 * EMBED_DIM), lambda r, h: (r, 0))],
        out_specs=pl.BlockSpec((EMBED_DIM, TR_ROWS), lambda r, h: (0, h * TR_NB + r)),
        out_shape=jax.ShapeDtypeStruct((EMBED_DIM, N_EDGES), jnp.float32),
        scratch_shapes=[pltpu.VMEM((2 * EMBED_DIM, TR_ROWS), jnp.float32)],
    )(x)


def kernel(edge_attr, weight):
    idx = edge_attr.astype(jnp.int32)
    pidx = idx[:NPAIR] * 4 + idx[NPAIR:]
    pidx_padded = jnp.zeros((NPAIR + 128,), jnp.int32).at[:NPAIR].set(pidx)
    w2 = jnp.concatenate(
        [jnp.repeat(weight, 4, axis=0), jnp.tile(weight, (4, 1))], axis=1
    )
    x = _sc_gather(pidx_padded, w2)
    y = _tc_transpose(x)
    return y.T


# speedup vs baseline: 1.7598x; 1.3666x over previous
import jax
import jax.numpy as jnp
from jax import lax
from jax.experimental import pallas as pl
from jax.experimental.pallas import tpu as pltpu
from jax.experimental.pallas import tpu_sc as plsc

N_EDGES = 800000
EMBED_DIM = 64
NPAIR = N_EDGES // 2
NUM_CORES = 2
NUM_SUBCORES = 16
NW = NUM_CORES * NUM_SUBCORES
PER_W = NPAIR // NW
STAGE = 12504
CHUNK = 200
NCHUNK = 64
NBUF = 4

TR_ROWS = 16000
TR_NB = NPAIR // TR_ROWS


def _sc_body(w2_hbm, idx_hbm, out_hbm, idx_v, table_s,
             b0, b1, b2, b3, g0, g1, g2, g3, w0, w1, w2, w3):
    cid = lax.axis_index("c")
    sid = lax.axis_index("s")
    wid = sid * NUM_CORES + cid

    raw = wid * PER_W
    base = raw + (raw % 8)
    nraw = (wid + 1) * PER_W
    nbase = nraw + (nraw % 8)
    size = nbase - base
    base = pl.multiple_of(base, 8)

    bufs = (b0, b1, b2, b3)
    gsems = (g0, g1, g2, g3)
    wsems = (w0, w1, w2, w3)

    @pl.when(sid == 0)
    def _stage_table():
        pltpu.sync_copy(w2_hbm, table_s)

    pltpu.sync_copy(idx_hbm.at[pl.ds(base, STAGE)], idx_v)
    plsc.subcore_barrier()

    def chunk_off(j):
        return pl.multiple_of(lax.min(j * CHUNK, size - CHUNK), 8)

    def start_gather(j, b):
        pltpu.async_copy(
            table_s.at[idx_v.at[pl.ds(chunk_off(j), CHUNK)]], bufs[b], gsems[b]
        )

    def wait_gather(b):
        pltpu.make_async_copy(
            table_s.at[idx_v.at[pl.ds(0, CHUNK)]], bufs[b], gsems[b]
        ).wait()

    def start_write(j, b):
        pltpu.async_copy(
            bufs[b], out_hbm.at[pl.ds(base + chunk_off(j), CHUNK)], wsems[b]
        )

    def wait_write(b):
        pltpu.make_async_copy(
            bufs[b], out_hbm.at[pl.ds(base, CHUNK)], wsems[b]
        ).wait()

    for b in range(NBUF):
        start_gather(b, b)

    def step(t, carry):
        for b in range(NBUF):
            wait_gather(b)
            start_write(NBUF * t + b, b)
        for b in range(NBUF):
            wait_write(b)
            start_gather(NBUF * t + NBUF + b, b)
        return carry

    lax.fori_loop(0, (NCHUNK - NBUF) // NBUF, step, 0)

    for b in range(NBUF):
        wait_gather(b)
        start_write(NCHUNK - NBUF + b, b)
    for b in range(NBUF):
        wait_write(b)


def _sc_gather(pidx_padded, w2):
    mesh = plsc.VectorSubcoreMesh(core_axis_name="c", subcore_axis_name="s")
    k = pl.kernel(
        _sc_body,
        out_type=jax.ShapeDtypeStruct((NPAIR, 2 * EMBED_DIM), jnp.float32),
        mesh=mesh,
        scratch_types=[
            pltpu.VMEM((STAGE,), jnp.int32),
            pltpu.VMEM_SHARED((16, 2 * EMBED_DIM), jnp.float32),
            pltpu.VMEM((CHUNK, 2 * EMBED_DIM), jnp.float32),
            pltpu.VMEM((CHUNK, 2 * EMBED_DIM), jnp.float32),
            pltpu.VMEM((CHUNK, 2 * EMBED_DIM), jnp.float32),
            pltpu.VMEM((CHUNK, 2 * EMBED_DIM), jnp.float32),
            pltpu.SemaphoreType.DMA,
            pltpu.SemaphoreType.DMA,
            pltpu.SemaphoreType.DMA,
            pltpu.SemaphoreType.DMA,
            pltpu.SemaphoreType.DMA,
            pltpu.SemaphoreType.DMA,
            pltpu.SemaphoreType.DMA,
            pltpu.SemaphoreType.DMA,
        ],
        compiler_params=pltpu.CompilerParams(
            use_tc_tiling_on_sc=False, needs_layout_passes=False
        ),
    )
    return k(w2, pidx_padded)


def _tr_body(x_ref, y_ref, t_ref):
    h = pl.program_id(1)

    @pl.when(h == 0)
    def _full_transpose():
        t_ref[...] = jnp.transpose(x_ref[...], (1, 0))

    y_ref[...] = t_ref[pl.ds(h * EMBED_DIM, EMBED_DIM), :]


def _tc_transpose(x):
    return pl.pallas_call(
        _tr_body,
        grid=(TR_NB, 2),
        in_specs=[pl.BlockSpec((TR_ROWS, 2 * EMBED_DIM), lambda r, h: (r, 0))],
        out_specs=pl.BlockSpec((EMBED_DIM, TR_ROWS), lambda r, h: (0, h * TR_NB + r)),
        out_shape=jax.ShapeDtypeStruct((EMBED_DIM, N_EDGES), jnp.float32),
        scratch_shapes=[pltpu.VMEM((2 * EMBED_DIM, TR_ROWS), jnp.float32)],
    )(x)


def kernel(edge_attr, weight):
    idx = edge_attr.astype(jnp.int32)
    pidx = idx[:NPAIR] * 4 + idx[NPAIR:]
    pidx_padded = jnp.zeros((NPAIR + 128,), jnp.int32).at[:NPAIR].set(pidx)
    w2 = jnp.concatenate(
        [jnp.repeat(weight, 4, axis=0), jnp.tile(weight, (4, 1))], axis=1
    )
    x = _sc_gather(pidx_padded, w2)
    y = _tc_transpose(x)
    return y.T
